# baseline (speedup 1.0000x reference)
import jax
import jax.numpy as jnp
from jax.experimental import pallas as pl
from jax.experimental.pallas import tpu as pltpu

_ROI_WEIGHT = 2.0
_NUM_KEEP = 64


def _frame_kernel(tok_ref, roi_ref, ws_ref, bs_ref, out_ref, oh_ref):
    block = tok_ref[0]
    n, d = block.shape
    cols = n // 8
    s = jnp.dot(block, ws_ref[:, :], preferred_element_type=jnp.float32)
    s = s.reshape(8, cols) + bs_ref[0, 0]
    bias = roi_ref[0].astype(jnp.float32) * (_ROI_WEIGHT - 1.0) + 1.0
    s = s * bias

    row_io = jax.lax.broadcasted_iota(jnp.int32, (8, cols), 0)
    col_io = jax.lax.broadcasted_iota(jnp.int32, (8, cols), 1)
    lin = row_io * cols + col_io
    lin_row = jax.lax.broadcasted_iota(jnp.int32, (1, n), 1)

    def body(k, s):
        m = jnp.max(s)
        idx = jnp.min(jnp.where(s == m, lin, n * 2))
        oh_ref[pl.ds(k, 1), :] = (lin_row == idx).astype(jnp.float32)
        return jnp.where(lin == idx, -jnp.inf, s)

    jax.lax.fori_loop(0, _NUM_KEEP, body, s)
    out_ref[0] = jnp.dot(oh_ref[:, :], block, preferred_element_type=jnp.float32)


def kernel(tokens, roi_mask, Ws, bs):
    B, T, N, D = tokens.shape
    F = B * T
    tok = tokens.reshape(F, N, D)
    roi = roi_mask.reshape(F, 8, N // 8)
    ws_t = Ws.reshape(D, 1)
    bs2 = bs.reshape(1, 1)

    out = pl.pallas_call(
        _frame_kernel,
        grid=(F,),
        in_specs=[
            pl.BlockSpec((1, N, D), lambda i: (i, 0, 0)),
            pl.BlockSpec((1, 8, N // 8), lambda i: (i, 0, 0)),
            pl.BlockSpec((D, 1), lambda i: (0, 0)),
            pl.BlockSpec((1, 1), lambda i: (0, 0)),
        ],
        out_specs=pl.BlockSpec((1, _NUM_KEEP, D), lambda i: (i, 0, 0)),
        out_shape=jax.ShapeDtypeStruct((F, _NUM_KEEP, D), jnp.float32),
        scratch_shapes=[pltpu.VMEM((_NUM_KEEP, N), jnp.float32)],
        compiler_params=pltpu.CompilerParams(
            dimension_semantics=("arbitrary",),
        ),
    )(tok, roi, ws_t, bs2)
    return out.reshape(B, T, _NUM_KEEP, D)
